# Initial kernel scaffold; baseline (speedup 1.0000x reference)
import jax
import jax.numpy as jnp
from jax.experimental import pallas as pl


def kernel(input_features, adj_edge_index, adj_values, W, b):
    raise NotImplementedError("write your pallas kernel here")


# speedup vs baseline: 4.1990x; 4.1990x over previous
import functools

import jax
import jax.numpy as jnp
from jax import lax
from jax.experimental import pallas as pl
from jax.experimental.pallas import tpu as pltpu
from jax.experimental.pallas import tpu_sc as plsc

N = 10000
E = 320000
D = 128
NC = 2
NS = 16
NW = NC * NS
CHUNK = 128
EPT = 10112
NCHUNK = EPT // CHUNK
E_PAD = EPT * NW
SLAB = 632
SLAB_LAST = N - SLAB * (NS - 1)
ACC_ROWS = SLAB * NS


def _mm_body(x_ref, w_ref, b_ref, o_ref):
    o_ref[...] = (
        lax.dot_general(
            x_ref[...], w_ref[...], (((1,), (1,)), ((), ())),
            preferred_element_type=jnp.float32,
        )
        + b_ref[...]
    )


def _linear(x, W, b2):
    rblk = 2000
    nr = N // rblk
    return pl.pallas_call(
        _mm_body,
        grid=(nr,),
        in_specs=[
            pl.BlockSpec((rblk, D), lambda r: (r, 0)),
            pl.BlockSpec((D, D), lambda r: (0, 0)),
            pl.BlockSpec((1, D), lambda r: (0, 0)),
        ],
        out_specs=pl.BlockSpec((rblk, D), lambda r: (r, 0)),
        out_shape=jax.ShapeDtypeStruct((N, D), jnp.float32),
    )(x, W, b2)


def _add_body(a_ref, b_ref, o_ref):
    o_ref[...] = a_ref[...] + b_ref[...]


def _combine(p):
    rblk = 2000
    nr = N // rblk
    return pl.pallas_call(
        _add_body,
        grid=(nr,),
        in_specs=[
            pl.BlockSpec((rblk, D), lambda r: (r, 0)),
            pl.BlockSpec((rblk, D), lambda r: (r + N // rblk, 0)),
        ],
        out_specs=pl.BlockSpec((rblk, D), lambda r: (r, 0)),
        out_shape=jax.ShapeDtypeStruct((N, D), jnp.float32),
    )(p, p)


def _sc_body(table, cols, rows, vals, out, cols_v, rows_v, vals_v, gbuf, accum):
    core = lax.axis_index("c")
    sid = lax.axis_index("s")
    wid = core * NS + sid

    pltpu.sync_copy(cols.at[wid], cols_v)
    pltpu.sync_copy(rows.at[wid], rows_v)
    pltpu.sync_copy(vals.at[wid], vals_v)

    zero16 = jnp.zeros((16,), jnp.float32)

    def _zrow(i, _):
        for f in range(D // 16):
            gbuf[i, pl.ds(f * 16, 16)] = zero16
        return _

    lax.fori_loop(0, CHUNK, _zrow, None, unroll=2)
    zbase = pl.multiple_of(sid * SLAB, 8)
    for off in range(0, SLAB, CHUNK):
        n = min(CHUNK, SLAB - off)
        pltpu.sync_copy(gbuf.at[pl.ds(0, n)],
                        accum.at[pl.ds(pl.multiple_of(zbase + off, 8), n)])
    plsc.subcore_barrier()

    def _chunk(j, _):
        pltpu.sync_copy(table.at[cols_v.at[j]], gbuf)

        def _edge(e, _c):
            jv = jnp.full((16,), j, jnp.int32)
            ev = jnp.full((16,), e, jnp.int32)
            splat = plsc.load_gather(vals_v, [jv, ev])
            for f in range(D // 16):
                g = gbuf[e, pl.ds(f * 16, 16)]
                gbuf[e, pl.ds(f * 16, 16)] = g * splat
            return _c

        lax.fori_loop(0, CHUNK, _edge, None, unroll=4)
        pltpu.sync_copy(gbuf, accum.at[rows_v.at[j]], add=True)
        return _

    lax.fori_loop(0, NCHUNK, _chunk, None)
    plsc.subcore_barrier()

    src = pl.multiple_of(sid * SLAB, 8)
    dst = pl.multiple_of(core * N + sid * SLAB, 8)

    @pl.when(sid < NS - 1)
    def _full():
        pltpu.sync_copy(accum.at[pl.ds(src, SLAB)], out.at[pl.ds(dst, SLAB)])

    @pl.when(sid == NS - 1)
    def _last():
        pltpu.sync_copy(accum.at[pl.ds(src, SLAB_LAST)],
                        out.at[pl.ds(dst, SLAB_LAST)])


@functools.cache
def _sc_aggregate():
    @functools.partial(
        pl.kernel,
        out_type=jax.ShapeDtypeStruct((NC * N, D), jnp.float32),
        mesh=plsc.VectorSubcoreMesh(core_axis_name="c", subcore_axis_name="s",
                                    num_cores=NC, num_subcores=NS),
        compiler_params=pltpu.CompilerParams(needs_layout_passes=False),
        scratch_types=[
            pltpu.VMEM((NCHUNK, CHUNK), jnp.int32),
            pltpu.VMEM((NCHUNK, CHUNK), jnp.int32),
            pltpu.VMEM((NCHUNK, CHUNK), jnp.float32),
            pltpu.VMEM((CHUNK, D), jnp.float32),
            pltpu.VMEM_SHARED((ACC_ROWS, D), jnp.float32),
        ],
    )
    def agg(table, cols, rows, vals, out, *scratch):
        _sc_body(table, cols, rows, vals, out, *scratch)

    return agg


def kernel(input_features, adj_edge_index, adj_values, W, b):
    table = _linear(input_features, W, b.reshape(1, D))

    col = adj_edge_index[1]
    row = adj_edge_index[0]
    pad = E_PAD - E
    col_p = jnp.pad(col, (0, pad))
    row_p = jnp.pad(row, (0, pad), constant_values=N)
    val_p = jnp.pad(adj_values, (0, pad))
    cols3 = col_p.reshape(NW, NCHUNK, CHUNK)
    rows3 = row_p.reshape(NW, NCHUNK, CHUNK)
    vals3 = val_p.reshape(NW, NCHUNK, CHUNK)

    partials = _sc_aggregate()(table, cols3, rows3, vals3)
    return _combine(partials)
